# baseline (speedup 1.0000x reference)
import functools

import jax
import jax.numpy as jnp
from jax import lax
from jax.experimental import pallas as pl
from jax.experimental.pallas import tpu as pltpu
from jax.experimental.pallas import tpu_sc as plsc

NC = 2
NS = 16
NW = NC * NS
LANES = 16
K = 100


def _sc_gather_sum(idx3, table, dummy, L, D):
    chunks = idx3.shape[1]
    half_steps = chunks // 2
    mesh = plsc.VectorSubcoreMesh(core_axis_name="c", subcore_axis_name="s")

    @functools.partial(
        pl.kernel,
        out_type=jax.ShapeDtypeStruct((NW, L, D), jnp.float32),
        mesh=mesh,
        scratch_types=[
            pltpu.VMEM((chunks, K), jnp.int32),
            pltpu.VMEM((K, D), jnp.float32),
            pltpu.VMEM((K, D), jnp.float32),
            pltpu.VMEM((L, D), jnp.float32),
            pltpu.SemaphoreType.DMA,
            pltpu.SemaphoreType.DMA,
        ],
    )
    def sc_kernel(idx_hbm, table_hbm, dummy_hbm, out_hbm, idx_v, buf0, buf1, acc, sem0, sem1):
        wid = lax.axis_index("s") * NC + lax.axis_index("c")
        pltpu.sync_copy(idx_hbm.at[wid], idx_v)

        zero = jnp.zeros((LANES,), jnp.float32)

        def zero_body(r, carry):
            for c in range(D // LANES):
                acc[r, pl.ds(c * LANES, LANES)] = zero
            return carry

        lax.fori_loop(0, L, zero_body, 0)

        def accumulate(buf, off):
            def body(r, carry):
                for c in range(D // LANES):
                    x = buf[r, pl.ds(c * LANES, LANES)]
                    plsc.addupdate(acc.at[off + r, pl.ds(c * LANES, LANES)], x)
                return carry
            lax.fori_loop(0, K, body, 0)

        def wait(buf, sem):
            pltpu.make_async_copy(dummy_hbm, buf, sem).wait()

        pltpu.async_copy(table_hbm.at[idx_v.at[0]], buf0, sem0)

        def step(jj, carry):
            j0 = 2 * jj
            pltpu.async_copy(table_hbm.at[idx_v.at[j0 + 1]], buf1, sem1)
            wait(buf0, sem0)
            accumulate(buf0, 0)

            @pl.when(jj < half_steps - 1)
            def _():
                pltpu.async_copy(table_hbm.at[idx_v.at[j0 + 2]], buf0, sem0)

            wait(buf1, sem1)
            accumulate(buf1, K)
            return carry

        lax.fori_loop(0, half_steps, step, 0)
        pltpu.sync_copy(acc, out_hbm.at[wid])

    return sc_kernel(idx3, table, dummy)


def _tc_project(partials, W, b2d, L, D, vocab):
    blk = 2048
    grid = pl.cdiv(vocab, blk)

    def body(p_ref, w_ref, b_ref, out_ref, s_ref):
        @pl.when(pl.program_id(0) == 0)
        def _():
            s_ref[...] = jnp.sum(p_ref[...], axis=0)

        out_ref[...] = lax.dot_general(
            s_ref[...].astype(jnp.bfloat16), w_ref[...].astype(jnp.bfloat16),
            (((1,), (1,)), ((), ())),
            preferred_element_type=jnp.float32,
        ) + b_ref[...]

    return pl.pallas_call(
        body,
        grid=(grid,),
        in_specs=[
            pl.BlockSpec((NW, L, D), lambda i: (0, 0, 0)),
            pl.BlockSpec((blk, D), lambda i: (i, 0)),
            pl.BlockSpec((1, blk), lambda i: (0, i)),
        ],
        out_specs=pl.BlockSpec((L, blk), lambda i: (0, i)),
        out_shape=jax.ShapeDtypeStruct((L, vocab), jnp.float32),
        scratch_shapes=[pltpu.VMEM((L, D), jnp.float32)],
    )(partials, W, b2d)


def kernel(inputs, emb_table, W, b):
    B, L = inputs.shape
    vocab, D = emb_table.shape
    chunks = B * L // (NW * K)
    idx3 = inputs.astype(jnp.int32).reshape(NW, chunks, K)
    dummy = jnp.zeros((K, D), jnp.float32)
    partials = _sc_gather_sum(idx3, emb_table, dummy, L, D)
    return _tc_project(partials, W, b.reshape(1, vocab), L, D, vocab)


# speedup vs baseline: 1.2102x; 1.2102x over previous
import functools

import jax
import jax.numpy as jnp
from jax import lax
from jax.experimental import pallas as pl
from jax.experimental.pallas import tpu as pltpu
from jax.experimental.pallas import tpu_sc as plsc

NC = 2
NS = 16
NW = NC * NS
LANES = 16
K = 100


def _sc_gather_sum(idx3, table, dummy, L, D):
    chunks = idx3.shape[1]
    mesh = plsc.VectorSubcoreMesh(core_axis_name="c", subcore_axis_name="s")

    nchain = 4
    rounds = chunks // nchain

    @functools.partial(
        pl.kernel,
        out_type=jax.ShapeDtypeStruct((NW, 2, K, D), jnp.float32),
        mesh=mesh,
        scratch_types=[
            pltpu.VMEM((chunks, K), jnp.int32),
            [pltpu.VMEM((K, D), jnp.float32)] * nchain,
            [pltpu.SemaphoreType.DMA] * nchain,
        ],
    )
    def sc_kernel(idx_hbm, table_hbm, dummy_hbm, out_hbm, idx_v, bufs, sems):
        wid = lax.axis_index("s") * NC + lax.axis_index("c")
        pltpu.sync_copy(idx_hbm.at[wid], idx_v)

        def wait(buf, sem):
            pltpu.make_async_copy(dummy_hbm, buf, sem).wait()

        for c in range(nchain):
            pltpu.async_copy(table_hbm.at[idx_v.at[c]], bufs[c], sems[c])

        def step(jj, carry):
            j0 = nchain * jj
            for c in range(nchain):
                wait(bufs[c], sems[c])
                pltpu.async_copy(
                    table_hbm.at[idx_v.at[j0 + c]], bufs[c], sems[c], add=True)
            return carry

        lax.fori_loop(1, rounds, step, 0)
        for c in range(nchain):
            wait(bufs[c], sems[c])

        @plsc.parallel_loop(0, K, 1, unroll=4)
        def _merge(r):
            for c in range(D // LANES):
                sl = pl.ds(c * LANES, LANES)
                plsc.addupdate(bufs[0].at[r, sl], bufs[2][r, sl])
                plsc.addupdate(bufs[1].at[r, sl], bufs[3][r, sl])

        pltpu.sync_copy(bufs[0], out_hbm.at[wid, 0])
        pltpu.sync_copy(bufs[1], out_hbm.at[wid, 1])

    return sc_kernel(idx3, table, dummy)


def _tc_project(partials, W, b2d, L, D, vocab):
    blk = 2048
    grid = pl.cdiv(vocab, blk)

    def body(p_ref, w_ref, b_ref, out_ref, s_ref):
        @pl.when(pl.program_id(0) == 0)
        def _():
            half = L // 2
            s_ref[pl.ds(0, half), :] = jnp.sum(p_ref[:, 0], axis=0)
            s_ref[pl.ds(half, half), :] = jnp.sum(p_ref[:, 1], axis=0)

        out_ref[...] = lax.dot_general(
            s_ref[...].astype(jnp.bfloat16), w_ref[...].astype(jnp.bfloat16),
            (((1,), (1,)), ((), ())),
            preferred_element_type=jnp.float32,
        ) + b_ref[...]

    return pl.pallas_call(
        body,
        grid=(grid,),
        in_specs=[
            pl.BlockSpec((NW, 2, L // 2, D), lambda i: (0, 0, 0, 0)),
            pl.BlockSpec((blk, D), lambda i: (i, 0)),
            pl.BlockSpec((1, blk), lambda i: (0, i)),
        ],
        out_specs=pl.BlockSpec((L, blk), lambda i: (0, i)),
        out_shape=jax.ShapeDtypeStruct((L, vocab), jnp.float32),
        scratch_shapes=[pltpu.VMEM((L, D), jnp.float32)],
    )(partials, W, b2d)


def kernel(inputs, emb_table, W, b):
    B, L = inputs.shape
    vocab, D = emb_table.shape
    chunks = B * L // (NW * K)
    idx3 = inputs.astype(jnp.int32).reshape(NW, chunks, K)
    dummy = jnp.zeros((K, D), jnp.float32)
    partials = _sc_gather_sum(idx3, emb_table, dummy, L, D)
    return _tc_project(partials, W, b.reshape(1, vocab), L, D, vocab)


# speedup vs baseline: 1.4085x; 1.1638x over previous
import functools

import jax
import jax.numpy as jnp
from jax import lax
from jax.experimental import pallas as pl
from jax.experimental.pallas import tpu as pltpu
from jax.experimental.pallas import tpu_sc as plsc

NC = 2
NS = 16
NW = NC * NS
LANES = 16
K = 100


def _sc_gather_sum(idx3, table, dummy, L, D):
    chunks = idx3.shape[1]
    mesh = plsc.VectorSubcoreMesh(core_axis_name="c", subcore_axis_name="s")

    nchain = 8
    rounds = chunks // nchain

    @functools.partial(
        pl.kernel,
        out_type=jax.ShapeDtypeStruct((NW, 2, K, D), jnp.float32),
        mesh=mesh,
        scratch_types=[
            pltpu.VMEM((chunks, K), jnp.int32),
            [pltpu.VMEM((K, D), jnp.float32)] * nchain,
            [pltpu.SemaphoreType.DMA] * nchain,
        ],
    )
    def sc_kernel(idx_hbm, table_hbm, dummy_hbm, out_hbm, idx_v, bufs, sems):
        wid = lax.axis_index("s") * NC + lax.axis_index("c")
        pltpu.sync_copy(idx_hbm.at[wid], idx_v)

        def wait(buf, sem):
            pltpu.make_async_copy(dummy_hbm, buf, sem).wait()

        for c in range(nchain):
            pltpu.async_copy(table_hbm.at[idx_v.at[c]], bufs[c], sems[c])

        def step(jj, carry):
            j0 = nchain * jj
            for c in range(nchain):
                wait(bufs[c], sems[c])
                pltpu.async_copy(
                    table_hbm.at[idx_v.at[j0 + c]], bufs[c], sems[c], add=True)
            return carry

        lax.fori_loop(1, rounds, step, 0)
        for c in range(nchain):
            wait(bufs[c], sems[c])

        @plsc.parallel_loop(0, K, 1, unroll=4)
        def _merge(r):
            for c in range(D // LANES):
                sl = pl.ds(c * LANES, LANES)
                for src in range(2, nchain, 2):
                    plsc.addupdate(bufs[0].at[r, sl], bufs[src][r, sl])
                    plsc.addupdate(bufs[1].at[r, sl], bufs[src + 1][r, sl])

        pltpu.sync_copy(bufs[0], out_hbm.at[wid, 0])
        pltpu.sync_copy(bufs[1], out_hbm.at[wid, 1])

    return sc_kernel(idx3, table, dummy)


def _tc_project(partials, W, b2d, L, D, vocab):
    blk = 4096
    grid = pl.cdiv(vocab, blk)

    def body(p_ref, w_ref, b_ref, out_ref, s_ref):
        @pl.when(pl.program_id(0) == 0)
        def _():
            half = L // 2
            s_ref[pl.ds(0, half), :] = jnp.sum(p_ref[:, 0], axis=0)
            s_ref[pl.ds(half, half), :] = jnp.sum(p_ref[:, 1], axis=0)

        out_ref[...] = lax.dot_general(
            s_ref[...].astype(jnp.bfloat16), w_ref[...].astype(jnp.bfloat16),
            (((1,), (1,)), ((), ())),
            preferred_element_type=jnp.float32,
        ) + b_ref[...]

    return pl.pallas_call(
        body,
        grid=(grid,),
        in_specs=[
            pl.BlockSpec((NW, 2, L // 2, D), lambda i: (0, 0, 0, 0)),
            pl.BlockSpec((blk, D), lambda i: (i, 0)),
            pl.BlockSpec((1, blk), lambda i: (0, i)),
        ],
        out_specs=pl.BlockSpec((L, blk), lambda i: (0, i)),
        out_shape=jax.ShapeDtypeStruct((L, vocab), jnp.float32),
        scratch_shapes=[pltpu.VMEM((L, D), jnp.float32)],
    )(partials, W, b2d)


def kernel(inputs, emb_table, W, b):
    B, L = inputs.shape
    vocab, D = emb_table.shape
    chunks = B * L // (NW * K)
    idx3 = inputs.astype(jnp.int32).reshape(NW, chunks, K)
    dummy = jnp.zeros((K, D), jnp.float32)
    partials = _sc_gather_sum(idx3, emb_table, dummy, L, D)
    return _tc_project(partials, W, b.reshape(1, vocab), L, D, vocab)


# speedup vs baseline: 1.4565x; 1.0341x over previous
import functools

import jax
import jax.numpy as jnp
from jax import lax
from jax.experimental import pallas as pl
from jax.experimental.pallas import tpu as pltpu
from jax.experimental.pallas import tpu_sc as plsc

NC = 2
NS = 16
NW = NC * NS
LANES = 16
K = 100


def _sc_gather_sum(idx3, table, dummy, L, D):
    chunks = idx3.shape[1]
    mesh = plsc.VectorSubcoreMesh(core_axis_name="c", subcore_axis_name="s")

    nchain = 8
    rounds = chunks // nchain

    @functools.partial(
        pl.kernel,
        out_type=jax.ShapeDtypeStruct((NW, 2, K, D), jnp.float32),
        mesh=mesh,
        scratch_types=[
            pltpu.VMEM((chunks, K), jnp.int32),
            [pltpu.VMEM((K, D), jnp.float32)] * nchain,
            [pltpu.SemaphoreType.DMA] * nchain,
        ],
    )
    def sc_kernel(idx_hbm, table_hbm, dummy_hbm, out_hbm, idx_v, bufs, sems):
        wid = lax.axis_index("s") * NC + lax.axis_index("c")
        pltpu.sync_copy(idx_hbm.at[wid], idx_v)

        def chunk_idx(j, c):
            return idx_v.at[j * nchain + c]

        def wait(buf, sem):
            pltpu.make_async_copy(dummy_hbm, buf, sem).wait()

        for c in range(nchain):
            pltpu.async_copy(table_hbm.at[chunk_idx(0, c)], bufs[c], sems[c])

        def step(jj, carry):
            for c in range(nchain):
                wait(bufs[c], sems[c])
                pltpu.async_copy(
                    table_hbm.at[chunk_idx(jj, c)], bufs[c], sems[c], add=True)
            return carry

        lax.fori_loop(1, rounds, step, 0)
        for c in range(nchain):
            wait(bufs[c], sems[c])

        @plsc.parallel_loop(0, K, 1, unroll=4)
        def _merge(r):
            for c in range(D // LANES):
                sl = pl.ds(c * LANES, LANES)
                for src in range(2, nchain, 2):
                    plsc.addupdate(bufs[0].at[r, sl], bufs[src][r, sl])
                    plsc.addupdate(bufs[1].at[r, sl], bufs[src + 1][r, sl])

        pltpu.sync_copy(bufs[0], out_hbm.at[wid, 0])
        pltpu.sync_copy(bufs[1], out_hbm.at[wid, 1])

    return sc_kernel(idx3, table, dummy)


def _tc_project(partials, W, b2d, L, D, vocab):
    blk = 8192
    grid = pl.cdiv(vocab, blk)

    def body(p_ref, w_ref, b_ref, out_ref, s_ref):
        @pl.when(pl.program_id(0) == 0)
        def _():
            half = L // 2
            s_ref[pl.ds(0, half), :] = jnp.sum(p_ref[:, 0], axis=0)
            s_ref[pl.ds(half, half), :] = jnp.sum(p_ref[:, 1], axis=0)

        out_ref[...] = lax.dot_general(
            s_ref[...].astype(jnp.bfloat16), w_ref[...].astype(jnp.bfloat16),
            (((1,), (1,)), ((), ())),
            preferred_element_type=jnp.float32,
        ) + b_ref[...]

    return pl.pallas_call(
        body,
        grid=(grid,),
        in_specs=[
            pl.BlockSpec((NW, 2, L // 2, D), lambda i: (0, 0, 0, 0)),
            pl.BlockSpec((blk, D), lambda i: (i, 0)),
            pl.BlockSpec((1, blk), lambda i: (0, i)),
        ],
        out_specs=pl.BlockSpec((L, blk), lambda i: (0, i)),
        out_shape=jax.ShapeDtypeStruct((L, vocab), jnp.float32),
        scratch_shapes=[pltpu.VMEM((L, D), jnp.float32)],
    )(partials, W, b2d)


def kernel(inputs, emb_table, W, b):
    B, L = inputs.shape
    vocab, D = emb_table.shape
    chunks = B * L // (NW * K)
    idx3 = inputs.astype(jnp.int32).reshape(NW, chunks, K)
    dummy = jnp.zeros((K, D), jnp.float32)
    partials = _sc_gather_sum(idx3, emb_table, dummy, L, D)
    return _tc_project(partials, W, b.reshape(1, vocab), L, D, vocab)


# speedup vs baseline: 1.4593x; 1.0019x over previous
import functools

import jax
import jax.numpy as jnp
from jax import lax
from jax.experimental import pallas as pl
from jax.experimental.pallas import tpu as pltpu
from jax.experimental.pallas import tpu_sc as plsc

NC = 2
NS = 16
NW = NC * NS
LANES = 16
K = 100


def _sc_gather_sum(idx3, table, dummy, L, D):
    chunks = idx3.shape[1]
    mesh = plsc.VectorSubcoreMesh(core_axis_name="c", subcore_axis_name="s")

    nchain = 8
    rounds = chunks // nchain

    @functools.partial(
        pl.kernel,
        out_type=jax.ShapeDtypeStruct((NW, 2, K, D), jnp.float32),
        mesh=mesh,
        scratch_types=[
            pltpu.VMEM((chunks, K), jnp.int32),
            [pltpu.VMEM((K, D), jnp.float32)] * nchain,
            [pltpu.SemaphoreType.DMA] * nchain,
        ],
    )
    def sc_kernel(idx_hbm, table_hbm, dummy_hbm, out_hbm, idx_v, bufs, sems):
        wid = lax.axis_index("s") * NC + lax.axis_index("c")
        pltpu.sync_copy(idx_hbm.at[wid], idx_v)

        def chunk_idx(j, c):
            return idx_v.at[j * nchain + c]

        def wait(buf, sem):
            pltpu.make_async_copy(dummy_hbm, buf, sem).wait()

        for c in range(nchain):
            pltpu.async_copy(table_hbm.at[chunk_idx(0, c)], bufs[c], sems[c])

        def step(jj, carry):
            for c in range(nchain):
                wait(bufs[c], sems[c])
                pltpu.async_copy(
                    table_hbm.at[chunk_idx(jj, c)], bufs[c], sems[c], add=True)
            return carry

        lax.fori_loop(1, rounds, step, 0)
        for c in range(nchain):
            wait(bufs[c], sems[c])

        @plsc.parallel_loop(0, K, 1, unroll=4)
        def _merge(r):
            for c in range(D // LANES):
                sl = pl.ds(c * LANES, LANES)
                for src in range(2, nchain, 2):
                    plsc.addupdate(bufs[0].at[r, sl], bufs[src][r, sl])
                    plsc.addupdate(bufs[1].at[r, sl], bufs[src + 1][r, sl])

        pltpu.sync_copy(bufs[0], out_hbm.at[wid, 0])
        pltpu.sync_copy(bufs[1], out_hbm.at[wid, 1])

    return sc_kernel(idx3, table, dummy)


def _tc_project(partials, W, b2d, L, D, vocab):
    blk = 10240
    grid = pl.cdiv(vocab, blk)

    def body(p_ref, w_ref, b_ref, out_ref, s_ref):
        @pl.when(pl.program_id(0) == 0)
        def _():
            half = L // 2
            s_ref[pl.ds(0, half), :] = jnp.sum(p_ref[:, 0], axis=0)
            s_ref[pl.ds(half, half), :] = jnp.sum(p_ref[:, 1], axis=0)

        out_ref[...] = lax.dot_general(
            s_ref[...].astype(jnp.bfloat16), w_ref[...].astype(jnp.bfloat16),
            (((1,), (1,)), ((), ())),
            preferred_element_type=jnp.float32,
        ) + b_ref[...]

    return pl.pallas_call(
        body,
        grid=(grid,),
        in_specs=[
            pl.BlockSpec((NW, 2, L // 2, D), lambda i: (0, 0, 0, 0)),
            pl.BlockSpec((blk, D), lambda i: (i, 0)),
            pl.BlockSpec((1, blk), lambda i: (0, i)),
        ],
        out_specs=pl.BlockSpec((L, blk), lambda i: (0, i)),
        out_shape=jax.ShapeDtypeStruct((L, vocab), jnp.float32),
        scratch_shapes=[pltpu.VMEM((L, D), jnp.float32)],
    )(partials, W, b2d)


def kernel(inputs, emb_table, W, b):
    B, L = inputs.shape
    vocab, D = emb_table.shape
    chunks = B * L // (NW * K)
    idx3 = inputs.astype(jnp.int32).reshape(NW, chunks, K)
    dummy = jnp.zeros((K, D), jnp.float32)
    partials = _sc_gather_sum(idx3, emb_table, dummy, L, D)
    return _tc_project(partials, W, b.reshape(1, vocab), L, D, vocab)


# speedup vs baseline: 1.4619x; 1.0018x over previous
import functools

import jax
import jax.numpy as jnp
from jax import lax
from jax.experimental import pallas as pl
from jax.experimental.pallas import tpu as pltpu
from jax.experimental.pallas import tpu_sc as plsc

NC = 2
NS = 16
NW = NC * NS
LANES = 16
K = 100


def _sc_gather_sum(idx3, table, dummy, L, D):
    chunks = idx3.shape[1]
    mesh = plsc.VectorSubcoreMesh(core_axis_name="c", subcore_axis_name="s")

    nchain = 8
    rounds = chunks // nchain

    @functools.partial(
        pl.kernel,
        out_type=jax.ShapeDtypeStruct((NW, 2, K, D), jnp.float32),
        mesh=mesh,
        scratch_types=[
            pltpu.VMEM((chunks, K), jnp.int32),
            [pltpu.VMEM((K, D), jnp.float32)] * nchain,
            [pltpu.SemaphoreType.DMA] * nchain,
        ],
    )
    def sc_kernel(idx_hbm, table_hbm, dummy_hbm, out_hbm, idx_v, bufs, sems):
        wid = lax.axis_index("s") * NC + lax.axis_index("c")
        pltpu.sync_copy(idx_hbm.at[wid], idx_v)

        def chunk_idx(j, c):
            return idx_v.at[j * nchain + c]

        def wait(buf, sem):
            pltpu.make_async_copy(dummy_hbm, buf, sem).wait()

        for c in range(nchain):
            pltpu.async_copy(table_hbm.at[chunk_idx(0, c)], bufs[c], sems[c])

        def step(jj, carry):
            for c in range(nchain):
                wait(bufs[c], sems[c])
                pltpu.async_copy(
                    table_hbm.at[chunk_idx(jj, c)], bufs[c], sems[c], add=True)
            return carry

        lax.fori_loop(1, rounds, step, 0)
        for c in range(nchain):
            wait(bufs[c], sems[c])

        @plsc.parallel_loop(0, K, 1, unroll=4)
        def _merge(r):
            for c in range(D // LANES):
                sl = pl.ds(c * LANES, LANES)
                for src in range(2, nchain, 2):
                    plsc.addupdate(bufs[0].at[r, sl], bufs[src][r, sl])
                    plsc.addupdate(bufs[1].at[r, sl], bufs[src + 1][r, sl])

        pltpu.sync_copy(bufs[0], out_hbm.at[wid, 0])
        pltpu.sync_copy(bufs[1], out_hbm.at[wid, 1])

    return sc_kernel(idx3, table, dummy)


def _tc_project(partials, W, b, L, D, vocab):
    blk = 10240
    grid = pl.cdiv(vocab, blk)

    def body(p_ref, w_ref, b_ref, out_ref, s_ref):
        @pl.when(pl.program_id(0) == 0)
        def _():
            half = L // 2
            s_ref[pl.ds(0, half), :] = jnp.sum(p_ref[:, 0], axis=0)
            s_ref[pl.ds(half, half), :] = jnp.sum(p_ref[:, 1], axis=0)

        out_ref[...] = lax.dot_general(
            s_ref[...].astype(jnp.bfloat16), w_ref[...].astype(jnp.bfloat16),
            (((1,), (1,)), ((), ())),
            preferred_element_type=jnp.float32,
        ) + b_ref[...][None, :]

    return pl.pallas_call(
        body,
        grid=(grid,),
        in_specs=[
            pl.BlockSpec((NW, 2, L // 2, D), lambda i: (0, 0, 0, 0)),
            pl.BlockSpec((blk, D), lambda i: (i, 0)),
            pl.BlockSpec((blk,), lambda i: (i,)),
        ],
        out_specs=pl.BlockSpec((L, blk), lambda i: (0, i)),
        out_shape=jax.ShapeDtypeStruct((L, vocab), jnp.float32),
        scratch_shapes=[pltpu.VMEM((L, D), jnp.float32)],
    )(partials, W, b)


def kernel(inputs, emb_table, W, b):
    B, L = inputs.shape
    vocab, D = emb_table.shape
    chunks = B * L // (NW * K)
    idx3 = inputs.astype(jnp.int32).reshape(NW, chunks, K)
    dummy = jnp.zeros((K, D), jnp.float32)
    partials = _sc_gather_sum(idx3, emb_table, dummy, L, D)
    return _tc_project(partials, W, b, L, D, vocab)
